# baseline (speedup 1.0000x reference)
import jax
import jax.numpy as jnp
from jax import lax
from jax.experimental import pallas as pl
from jax.experimental.pallas import tpu as pltpu
from jax.experimental.pallas import tpu_sc as plsc

B, L, H, Dn, Dh = 16, 2048, 24, 256, 100
LANES = 16
HALF = L // 2
CHT = 128
NCH = HALF // CHT
CW = 128

MASK_SC = -9999999999.0
MASK_WC = -99999999999.0
MASK_WV = -100000000000.0


def _body(wn, l_n_h, wh_h, l_hs_h, cls_h, g_sc_h, g_wc_h,
          o_sc, o_sa, o_wn, o_wc, o_wo, o_wv,
          gbuf, obuf, whb, clsb, lnb, lhsb, gscb, gwcb,
          scb, sab, wnb, wcb, wob, sem):
    b = lax.axis_index("s")
    h = lax.axis_index("c")
    iota = lax.iota(jnp.int32, LANES)
    l0 = h * HALF

    pltpu.sync_copy(l_n_h, lnb)
    pltpu.sync_copy(g_wc_h, gwcb)

    def start(ch):
        return pltpu.async_copy(
            wn.at[b, pl.ds(l0 + ch * CHT, CHT), pl.ds(0, CW)],
            gbuf.at[ch % 2], sem)

    copies = {0: start(0), 1: start(1)}

    @pl.when(jnp.logical_and(b == 0, h == 0))
    def _small():
        pltpu.sync_copy(wh_h, whb)
        pltpu.sync_copy(cls_h, clsb)
        pltpu.sync_copy(l_hs_h, lhsb)
        pltpu.sync_copy(g_sc_h, gscb)
        lhs_v = lhsb[...]
        base_b = iota * (H * Dh)
        for j in range(H):
            hm = jnp.int32(j) >= lhs_v
            v0 = plsc.load_gather(whb, [base_b + (j * Dh + 0)])
            plsc.store_scatter(scb, [iota * H + j], jnp.where(hm, MASK_SC, v0))
            v8 = plsc.load_gather(whb, [base_b + (j * Dh + 8)])
            plsc.store_scatter(wcb, [iota * H + j], jnp.where(hm, MASK_WC, v8))
        gsc_v = gscb[...]
        for j in range(6):
            v = plsc.load_gather(whb, [base_b + gsc_v * Dh + (1 + j)])
            plsc.store_scatter(sab, [iota * 6 + j], v)
        for j in range(5):
            v = plsc.load_gather(clsb, [iota * Dh + j])
            plsc.store_scatter(wnb, [iota * 5 + j], v)
        for w in range(4):
            cw = plsc.load_gather(gwcb, [iota * 4 + w])
            for j in range(4):
                v = plsc.load_gather(whb, [base_b + cw * Dh + (10 + j)])
                plsc.store_scatter(wob, [iota * 16 + (w * 4 + j)], v)
        pltpu.sync_copy(scb, o_sc)
        pltpu.sync_copy(sab, o_sa)
        pltpu.sync_copy(wnb, o_wn)
        pltpu.sync_copy(wcb, o_wc)
        pltpu.sync_copy(wob, o_wo)

    ln_b = plsc.load_gather(lnb, [jnp.full((LANES,), b, jnp.int32)])
    chans = []
    for w in range(4):
        c0 = plsc.load_gather(gwcb, [jnp.full((LANES,), b * 4 + w, jnp.int32)])
        chans.append((c0, c0 + 100))

    for ch in range(NCH):
        copies.pop(ch).wait()
        cur = jnp.full((LANES,), ch % 2, jnp.int32)
        for w in range(4):
            for k in range(2):
                cvec = chans[w][k]

                def grp(m, _, cvec=cvec, w=w, k=k, cur=cur, ch=ch):
                    l_loc = m * LANES + iota
                    vals = plsc.load_gather(gbuf, [cur, l_loc, cvec])
                    nm = (l0 + ch * CHT + l_loc) >= ln_b
                    vals = jnp.where(nm, MASK_WV, vals)
                    plsc.store_scatter(
                        obuf, [(w * HALF + ch * CHT + l_loc) * 2 + k], vals)
                    return 0

                lax.fori_loop(0, CHT // LANES, grp, 0)
        if ch + 2 < NCH:
            copies[ch + 2] = start(ch + 2)

    for w in range(4):
        pltpu.sync_copy(
            obuf.at[pl.ds(w * 2 * HALF, 2 * HALF)],
            o_wv.at[pl.ds(((b * 4 + w) * L + h * HALF) * 2, 2 * HALF)])


def _sc_call(wn, l_n, wh, l_hs, cls, g_sc, g_wc):
    return pl.kernel(
        _body,
        out_type=[
            jax.ShapeDtypeStruct((B * H,), jnp.float32),
            jax.ShapeDtypeStruct((B * 6,), jnp.float32),
            jax.ShapeDtypeStruct((B * 5,), jnp.float32),
            jax.ShapeDtypeStruct((B * H,), jnp.float32),
            jax.ShapeDtypeStruct((B * 16,), jnp.float32),
            jax.ShapeDtypeStruct((B * 4 * L * 2,), jnp.float32),
        ],
        mesh=plsc.VectorSubcoreMesh(core_axis_name="c", subcore_axis_name="s"),
        compiler_params=pltpu.CompilerParams(needs_layout_passes=False),
        scratch_types=[
            pltpu.VMEM((2, CHT, CW), jnp.float32),
            pltpu.VMEM((4 * HALF * 2,), jnp.float32),
            pltpu.VMEM((B * H * Dh,), jnp.float32),
            pltpu.VMEM((B * Dh,), jnp.float32),
            pltpu.VMEM((B,), jnp.int32),
            pltpu.VMEM((B,), jnp.int32),
            pltpu.VMEM((B,), jnp.int32),
            pltpu.VMEM((B * 4,), jnp.int32),
            pltpu.VMEM((B * H,), jnp.float32),
            pltpu.VMEM((B * 6,), jnp.float32),
            pltpu.VMEM((B * 5,), jnp.float32),
            pltpu.VMEM((B * H,), jnp.float32),
            pltpu.VMEM((B * 16,), jnp.float32),
            pltpu.SemaphoreType.DMA,
        ],
    )(wn, l_n, wh, l_hs, cls, g_sc, g_wc)


def kernel(wemb_n, l_n, wemb_h, l_hs, cls_vec, g_sc, g_sa, g_wn, g_wc, g_wo):
    o_sc, o_sa, o_wn, o_wc, o_wo, o_wv = _sc_call(
        wemb_n,
        l_n.astype(jnp.int32),
        wemb_h.reshape(B * H * Dh),
        l_hs.astype(jnp.int32),
        cls_vec.reshape(B * Dh),
        g_sc.astype(jnp.int32),
        g_wc.reshape(B * 4).astype(jnp.int32),
    )
    return (o_sc.reshape(B, H), o_sa.reshape(B, 6), o_wn.reshape(B, 5),
            o_wc.reshape(B, H), o_wo.reshape(B, 4, 4),
            o_wv.reshape(B, 4, L, 2))


# speedup vs baseline: 3.0563x; 3.0563x over previous
import jax
import jax.numpy as jnp
from jax import lax
from jax.experimental import pallas as pl
from jax.experimental.pallas import tpu as pltpu
from jax.experimental.pallas import tpu_sc as plsc

B, L, H, Dn, Dh = 16, 2048, 24, 256, 100
LANES = 16
HALF = L // 2
CHT = 128
NCH = HALF // CHT
CW = 128

MASK_SC = -9999999999.0
MASK_WC = -99999999999.0
MASK_WV = -100000000000.0


def _body(wn, l_n_h, wh_h, l_hs_h, cls_h, g_sc_h, g_wc_h,
          o_sc, o_sa, o_wn, o_wc, o_wo, o_wv,
          gbuf, obuf, whb, clsb, lnb, lhsb, gscb, gwcb,
          scb, sab, wnb, wcb, wob, sem):
    b = lax.axis_index("s")
    h = lax.axis_index("c")
    iota = lax.iota(jnp.int32, LANES)
    l0 = h * HALF

    pltpu.sync_copy(l_n_h, lnb)
    pltpu.sync_copy(g_wc_h, gwcb)

    def start(ch):
        return pltpu.async_copy(
            wn.at[b, pl.ds(l0 + ch * CHT, CHT), pl.ds(0, CW)],
            gbuf.at[ch % 2], sem)

    copies = {0: start(0), 1: start(1)}

    @pl.when(jnp.logical_and(b == 0, h == 0))
    def _small():
        pltpu.sync_copy(wh_h, whb)
        pltpu.sync_copy(cls_h, clsb)
        pltpu.sync_copy(l_hs_h, lhsb)
        pltpu.sync_copy(g_sc_h, gscb)
        lhs_v = lhsb[...]
        base_b = iota * (H * Dh)
        for j in range(H):
            hm = jnp.int32(j) >= lhs_v
            v0 = plsc.load_gather(whb, [base_b + (j * Dh + 0)])
            plsc.store_scatter(scb, [iota * H + j], jnp.where(hm, MASK_SC, v0))
            v8 = plsc.load_gather(whb, [base_b + (j * Dh + 8)])
            plsc.store_scatter(wcb, [iota * H + j], jnp.where(hm, MASK_WC, v8))
        gsc_v = gscb[...]
        for j in range(6):
            v = plsc.load_gather(whb, [base_b + gsc_v * Dh + (1 + j)])
            plsc.store_scatter(sab, [iota * 6 + j], v)
        for j in range(5):
            v = plsc.load_gather(clsb, [iota * Dh + j])
            plsc.store_scatter(wnb, [iota * 5 + j], v)
        for w in range(4):
            cw = plsc.load_gather(gwcb, [iota * 4 + w])
            for j in range(4):
                v = plsc.load_gather(whb, [base_b + cw * Dh + (10 + j)])
                plsc.store_scatter(wob, [iota * 16 + (w * 4 + j)], v)
        pltpu.sync_copy(scb, o_sc)
        pltpu.sync_copy(sab, o_sa)
        pltpu.sync_copy(wnb, o_wn)
        pltpu.sync_copy(wcb, o_wc)
        pltpu.sync_copy(wob, o_wo)

    ln_b = plsc.load_gather(lnb, [jnp.full((LANES,), b, jnp.int32)])
    chans = []
    for w in range(4):
        c0 = plsc.load_gather(gwcb, [jnp.full((LANES,), b * 4 + w, jnp.int32)])
        chans.append((c0, c0 + 100))

    for ch in range(NCH):
        copies.pop(ch).wait()
        cur = jnp.full((LANES,), ch % 2, jnp.int32)
        for w in range(4):
            for k in range(2):
                cvec = chans[w][k]

                def grp(m, _, cvec=cvec, w=w, k=k, cur=cur, ch=ch):
                    l_loc = m * LANES + iota
                    vals = plsc.load_gather(gbuf, [cur, l_loc, cvec])
                    nm = (l0 + ch * CHT + l_loc) >= ln_b
                    vals = jnp.where(nm, MASK_WV, vals)
                    obuf[w, ch, k, pl.ds(m * LANES, LANES)] = vals
                    return 0

                lax.fori_loop(0, CHT // LANES, grp, 0)
        if ch + 2 < NCH:
            copies[ch + 2] = start(ch + 2)

    for w in range(4):
        pltpu.sync_copy(obuf.at[w], o_wv.at[b * 4 + w, pl.ds(h * NCH, NCH)])


def _sc_call(wn, l_n, wh, l_hs, cls, g_sc, g_wc):
    return pl.kernel(
        _body,
        out_type=[
            jax.ShapeDtypeStruct((B * H,), jnp.float32),
            jax.ShapeDtypeStruct((B * 6,), jnp.float32),
            jax.ShapeDtypeStruct((B * 5,), jnp.float32),
            jax.ShapeDtypeStruct((B * H,), jnp.float32),
            jax.ShapeDtypeStruct((B * 16,), jnp.float32),
            jax.ShapeDtypeStruct((B * 4, L // CHT, 2, CHT), jnp.float32),
        ],
        mesh=plsc.VectorSubcoreMesh(core_axis_name="c", subcore_axis_name="s"),
        compiler_params=pltpu.CompilerParams(needs_layout_passes=False),
        scratch_types=[
            pltpu.VMEM((2, CHT, CW), jnp.float32),
            pltpu.VMEM((4, NCH, 2, CHT), jnp.float32),
            pltpu.VMEM((B * H * Dh,), jnp.float32),
            pltpu.VMEM((B * Dh,), jnp.float32),
            pltpu.VMEM((B,), jnp.int32),
            pltpu.VMEM((B,), jnp.int32),
            pltpu.VMEM((B,), jnp.int32),
            pltpu.VMEM((B * 4,), jnp.int32),
            pltpu.VMEM((B * H,), jnp.float32),
            pltpu.VMEM((B * 6,), jnp.float32),
            pltpu.VMEM((B * 5,), jnp.float32),
            pltpu.VMEM((B * H,), jnp.float32),
            pltpu.VMEM((B * 16,), jnp.float32),
            pltpu.SemaphoreType.DMA,
        ],
    )(wn, l_n, wh, l_hs, cls, g_sc, g_wc)


def kernel(wemb_n, l_n, wemb_h, l_hs, cls_vec, g_sc, g_sa, g_wn, g_wc, g_wo):
    o_sc, o_sa, o_wn, o_wc, o_wo, o_wv = _sc_call(
        wemb_n,
        l_n.astype(jnp.int32),
        wemb_h.reshape(B * H * Dh),
        l_hs.astype(jnp.int32),
        cls_vec.reshape(B * Dh),
        g_sc.astype(jnp.int32),
        g_wc.reshape(B * 4).astype(jnp.int32),
    )
    s_wv = (o_wv.reshape(B, 4, L // CHT, 2, CHT)
            .transpose(0, 1, 2, 4, 3)
            .reshape(B, 4, L, 2))
    return (o_sc.reshape(B, H), o_sa.reshape(B, 6), o_wn.reshape(B, 5),
            o_wc.reshape(B, H), o_wo.reshape(B, 4, 4), s_wv)


# speedup vs baseline: 3.9359x; 1.2878x over previous
import jax
import jax.numpy as jnp
from jax import lax
from jax.experimental import pallas as pl
from jax.experimental.pallas import tpu as pltpu
from jax.experimental.pallas import tpu_sc as plsc

B, L, H, Dn, Dh = 16, 2048, 24, 256, 100
LANES = 16
HALF = L // 2
CHT = 128
NCH = HALF // CHT
CW = 128

MASK_SC = -9999999999.0
MASK_WC = -99999999999.0
MASK_WV = -100000000000.0


def _full(v):
    return jnp.full((LANES,), v, jnp.int32)


def _body(wn, l_n_h, wh_h, l_hs_h, cls_h, g_sc_h, g_wc_h,
          o_sc, o_sa, o_wn, o_wc, o_wo, o_wv,
          gbuf, obuf, whb, clsb, lnb, lhsb, gscb, gwcb,
          scb, sab, wnb, wcb, wob, sem):
    b = lax.axis_index("s")
    h = lax.axis_index("c")
    iota = lax.iota(jnp.int32, LANES)
    l0 = h * HALF

    pltpu.sync_copy(l_n_h, lnb)
    pltpu.sync_copy(g_wc_h, gwcb)

    def start(ch):
        return pltpu.async_copy(
            wn.at[b, pl.ds(l0 + ch * CHT, CHT), pl.ds(0, CW)],
            gbuf.at[ch % 2], sem)

    copies = {0: start(0), 1: start(1)}

    @pl.when(jnp.logical_and(b == 0, h == 0))
    def _small0():
        pltpu.sync_copy(wh_h, whb)
        pltpu.sync_copy(l_hs_h, lhsb)
        lhs_v = lhsb[...]
        for j in range(H):
            hm = jnp.int32(j) >= lhs_v
            v0 = plsc.load_gather(whb, [iota, _full(j), _full(0)])
            plsc.store_scatter(scb, [iota, _full(j)], jnp.where(hm, MASK_SC, v0))
            v8 = plsc.load_gather(whb, [iota, _full(j), _full(8)])
            plsc.store_scatter(wcb, [iota, _full(j)], jnp.where(hm, MASK_WC, v8))
        pltpu.sync_copy(scb, o_sc)
        pltpu.sync_copy(wcb, o_wc)

    @pl.when(jnp.logical_and(b == 0, h == 1))
    def _small1():
        pltpu.sync_copy(wh_h, whb)
        pltpu.sync_copy(cls_h, clsb)
        pltpu.sync_copy(g_sc_h, gscb)
        gsc_v = gscb[...]
        for j in range(6):
            v = plsc.load_gather(whb, [iota, gsc_v, _full(1 + j)])
            sab[j, pl.ds(0, LANES)] = v
        for j in range(5):
            v = plsc.load_gather(clsb, [iota, _full(j)])
            wnb[j, pl.ds(0, LANES)] = v
        for w in range(4):
            cw = plsc.load_gather(gwcb, [_full(w), iota])
            for j in range(4):
                v = plsc.load_gather(whb, [iota, cw, _full(10 + j)])
                wob[w, j, pl.ds(0, LANES)] = v
        pltpu.sync_copy(sab, o_sa)
        pltpu.sync_copy(wnb, o_wn)
        pltpu.sync_copy(wob, o_wo)

    ln_b = plsc.load_gather(lnb, [_full(b)])
    chans = []
    for w in range(4):
        c0 = plsc.load_gather(gwcb, [_full(w), _full(b)])
        chans.append((c0, c0 + 100))

    for ch in range(NCH):
        copies.pop(ch).wait()
        cur = _full(ch % 2)

        def mloop(m, _, cur=cur, ch=ch):
            l_loc = m * LANES + iota
            nm = (l0 + ch * CHT + l_loc) >= ln_b
            for w in range(4):
                for k in range(2):
                    vals = plsc.load_gather(gbuf, [cur, l_loc, chans[w][k]])
                    vals = jnp.where(nm, MASK_WV, vals)
                    obuf[w, ch, k, pl.ds(m * LANES, LANES)] = vals
            return 0

        lax.fori_loop(0, CHT // LANES, mloop, 0)
        if ch + 2 < NCH:
            copies[ch + 2] = start(ch + 2)

    for w in range(4):
        pltpu.sync_copy(obuf.at[w], o_wv.at[b * 4 + w, pl.ds(h * NCH, NCH)])


def _sc_call(wn, l_n, wh, l_hs, cls, g_sc, g_wc_t):
    return pl.kernel(
        _body,
        out_type=[
            jax.ShapeDtypeStruct((B, H), jnp.float32),
            jax.ShapeDtypeStruct((6, B), jnp.float32),
            jax.ShapeDtypeStruct((5, B), jnp.float32),
            jax.ShapeDtypeStruct((B, H), jnp.float32),
            jax.ShapeDtypeStruct((4, 4, B), jnp.float32),
            jax.ShapeDtypeStruct((B * 4, L // CHT, 2, CHT), jnp.float32),
        ],
        mesh=plsc.VectorSubcoreMesh(core_axis_name="c", subcore_axis_name="s"),
        compiler_params=pltpu.CompilerParams(needs_layout_passes=False),
        scratch_types=[
            pltpu.VMEM((2, CHT, CW), jnp.float32),
            pltpu.VMEM((4, NCH, 2, CHT), jnp.float32),
            pltpu.VMEM((B, H, Dh), jnp.float32),
            pltpu.VMEM((B, Dh), jnp.float32),
            pltpu.VMEM((B,), jnp.int32),
            pltpu.VMEM((B,), jnp.int32),
            pltpu.VMEM((B,), jnp.int32),
            pltpu.VMEM((4, B), jnp.int32),
            pltpu.VMEM((B, H), jnp.float32),
            pltpu.VMEM((6, B), jnp.float32),
            pltpu.VMEM((5, B), jnp.float32),
            pltpu.VMEM((B, H), jnp.float32),
            pltpu.VMEM((4, 4, B), jnp.float32),
            pltpu.SemaphoreType.DMA,
        ],
    )(wn, l_n, wh, l_hs, cls, g_sc, g_wc_t)


def kernel(wemb_n, l_n, wemb_h, l_hs, cls_vec, g_sc, g_sa, g_wn, g_wc, g_wo):
    o_sc, o_sa, o_wn, o_wc, o_wo, o_wv = _sc_call(
        wemb_n,
        l_n.astype(jnp.int32),
        wemb_h,
        l_hs.astype(jnp.int32),
        cls_vec,
        g_sc.astype(jnp.int32),
        g_wc.astype(jnp.int32).T,
    )
    s_wv = (o_wv.reshape(B, 4, L // CHT, 2, CHT)
            .transpose(0, 1, 2, 4, 3)
            .reshape(B, 4, L, 2))
    return (o_sc, o_sa.T, o_wn.T, o_wc, jnp.transpose(o_wo, (2, 0, 1)), s_wv)


# speedup vs baseline: 4.1254x; 1.0481x over previous
import jax
import jax.numpy as jnp
from jax import lax
from jax.experimental import pallas as pl
from jax.experimental.pallas import tpu as pltpu
from jax.experimental.pallas import tpu_sc as plsc

B, L, H, Dn, Dh = 16, 2048, 24, 256, 100
LANES = 16
HALF = L // 2
CHT = 128
NCH = HALF // CHT
CW = 128
NBUF = 3

MASK_SC = -9999999999.0
MASK_WC = -99999999999.0
MASK_WV = -100000000000.0


def _full(v):
    return jnp.full((LANES,), v, jnp.int32)


def _body(wn, l_n_h, wh_h, l_hs_h, cls_h, g_sc_h, g_wc_h,
          o_sc, o_sa, o_wn, o_wc, o_wo, o_wv,
          gbuf, obuf, whb, clsb, lnb, lhsb, gscb, gwcb,
          scb, sab, wnb, wcb, wob, sem, sem2):
    b = lax.axis_index("s")
    h = lax.axis_index("c")
    iota = lax.iota(jnp.int32, LANES)
    l0 = h * HALF

    def start(ch):
        return pltpu.async_copy(
            wn.at[b, pl.ds(l0 + ch * CHT, CHT), pl.ds(0, CW)],
            gbuf.at[ch % NBUF], sem)

    copies = {ch: start(ch) for ch in range(NBUF)}

    pltpu.sync_copy(l_n_h, lnb)
    pltpu.sync_copy(g_wc_h, gwcb)

    @pl.when(jnp.logical_and(b == 0, h == 0))
    def _small0():
        st_w = pltpu.async_copy(wh_h, whb, sem2)
        pltpu.sync_copy(l_hs_h, lhsb)
        st_w.wait()
        lhs_v = lhsb[...]
        for j in range(H):
            hm = jnp.int32(j) >= lhs_v
            v0 = plsc.load_gather(whb, [iota, _full(j), _full(0)])
            plsc.store_scatter(scb, [iota, _full(j)], jnp.where(hm, MASK_SC, v0))
            v8 = plsc.load_gather(whb, [iota, _full(j), _full(8)])
            plsc.store_scatter(wcb, [iota, _full(j)], jnp.where(hm, MASK_WC, v8))
        pltpu.sync_copy(scb, o_sc)
        pltpu.sync_copy(wcb, o_wc)

    @pl.when(jnp.logical_and(b == 0, h == 1))
    def _small1():
        st_w = pltpu.async_copy(wh_h, whb, sem2)
        pltpu.sync_copy(cls_h, clsb)
        pltpu.sync_copy(g_sc_h, gscb)
        st_w.wait()
        gsc_v = gscb[...]
        for j in range(6):
            v = plsc.load_gather(whb, [iota, gsc_v, _full(1 + j)])
            sab[j, pl.ds(0, LANES)] = v
        for j in range(5):
            v = plsc.load_gather(clsb, [iota, _full(j)])
            wnb[j, pl.ds(0, LANES)] = v
        for w in range(4):
            cw = plsc.load_gather(gwcb, [_full(w), iota])
            for j in range(4):
                v = plsc.load_gather(whb, [iota, cw, _full(10 + j)])
                wob[w, j, pl.ds(0, LANES)] = v
        pltpu.sync_copy(sab, o_sa)
        pltpu.sync_copy(wnb, o_wn)
        pltpu.sync_copy(wob, o_wo)

    ln_b = plsc.load_gather(lnb, [_full(b)])
    chans = []
    for w in range(4):
        c0 = plsc.load_gather(gwcb, [_full(w), _full(b)])
        chans.append((c0, c0 + 100))

    for ch in range(NCH):
        copies.pop(ch).wait()
        cur = _full(ch % NBUF)

        def mloop(m, _, cur=cur, ch=ch):
            l_loc = m * LANES + iota
            nm = (l0 + ch * CHT + l_loc) >= ln_b
            for w in range(4):
                for k in range(2):
                    vals = plsc.load_gather(gbuf, [cur, l_loc, chans[w][k]])
                    vals = jnp.where(nm, MASK_WV, vals)
                    obuf[w, ch, k, pl.ds(m * LANES, LANES)] = vals
            return 0

        lax.fori_loop(0, CHT // LANES, mloop, 0)
        if ch + NBUF < NCH:
            copies[ch + NBUF] = start(ch + NBUF)

    writes = [
        pltpu.async_copy(obuf.at[w], o_wv.at[b * 4 + w, pl.ds(h * NCH, NCH)],
                         sem2)
        for w in range(4)
    ]
    for c in writes:
        c.wait()


def _sc_call(wn, l_n, wh, l_hs, cls, g_sc, g_wc_t):
    return pl.kernel(
        _body,
        out_type=[
            jax.ShapeDtypeStruct((B, H), jnp.float32),
            jax.ShapeDtypeStruct((6, B), jnp.float32),
            jax.ShapeDtypeStruct((5, B), jnp.float32),
            jax.ShapeDtypeStruct((B, H), jnp.float32),
            jax.ShapeDtypeStruct((4, 4, B), jnp.float32),
            jax.ShapeDtypeStruct((B * 4, L // CHT, 2, CHT), jnp.float32),
        ],
        mesh=plsc.VectorSubcoreMesh(core_axis_name="c", subcore_axis_name="s"),
        compiler_params=pltpu.CompilerParams(needs_layout_passes=False),
        scratch_types=[
            pltpu.VMEM((NBUF, CHT, CW), jnp.float32),
            pltpu.VMEM((4, NCH, 2, CHT), jnp.float32),
            pltpu.VMEM((B, H, Dh), jnp.float32),
            pltpu.VMEM((B, Dh), jnp.float32),
            pltpu.VMEM((B,), jnp.int32),
            pltpu.VMEM((B,), jnp.int32),
            pltpu.VMEM((B,), jnp.int32),
            pltpu.VMEM((4, B), jnp.int32),
            pltpu.VMEM((B, H), jnp.float32),
            pltpu.VMEM((6, B), jnp.float32),
            pltpu.VMEM((5, B), jnp.float32),
            pltpu.VMEM((B, H), jnp.float32),
            pltpu.VMEM((4, 4, B), jnp.float32),
            pltpu.SemaphoreType.DMA,
            pltpu.SemaphoreType.DMA,
        ],
    )(wn, l_n, wh, l_hs, cls, g_sc, g_wc_t)


def kernel(wemb_n, l_n, wemb_h, l_hs, cls_vec, g_sc, g_sa, g_wn, g_wc, g_wo):
    o_sc, o_sa, o_wn, o_wc, o_wo, o_wv = _sc_call(
        wemb_n,
        l_n.astype(jnp.int32),
        wemb_h,
        l_hs.astype(jnp.int32),
        cls_vec,
        g_sc.astype(jnp.int32),
        g_wc.astype(jnp.int32).T,
    )
    s_wv = (o_wv.reshape(B, 4, L // CHT, 2, CHT)
            .transpose(0, 1, 2, 4, 3)
            .reshape(B, 4, L, 2))
    return (o_sc, o_sa.T, o_wn.T, o_wc, jnp.transpose(o_wo, (2, 0, 1)), s_wv)


# speedup vs baseline: 4.1558x; 1.0074x over previous
import jax
import jax.numpy as jnp
from jax import lax
from jax.experimental import pallas as pl
from jax.experimental.pallas import tpu as pltpu
from jax.experimental.pallas import tpu_sc as plsc

B, L, H, Dn, Dh = 16, 2048, 24, 256, 100
LANES = 16
HALF = L // 2
CHT = 128
NCH = HALF // CHT
CW = 128
NBUF = 3

MASK_SC = -9999999999.0
MASK_WC = -99999999999.0
MASK_WV = -100000000000.0


def _full(v):
    return jnp.full((LANES,), v, jnp.int32)


def _body(wn, l_n_h, wh_h, l_hs_h, cls_h, g_sc_h, g_wc_h,
          o_sc, o_sa, o_wn, o_wc, o_wo, o_wv,
          gbuf, obuf, whb, clsb, lnb, lhsb, gscb, gwcb,
          scb, sab, wnb, wcb, wob, sem, sem2):
    b = lax.axis_index("s")
    h = lax.axis_index("c")
    iota = lax.iota(jnp.int32, LANES)
    l0 = h * HALF

    def chunk_refs(ch):
        return (wn.at[b, pl.ds(l0 + ch * CHT, CHT), pl.ds(0, CW)],
                gbuf.at[ch % NBUF])

    def start(ch):
        s, d = chunk_refs(ch)
        pltpu.async_copy(s, d, sem)

    pltpu.sync_copy(l_n_h, lnb)
    pltpu.sync_copy(g_wc_h, gwcb)
    ln_b = plsc.load_gather(lnb, [_full(b)])

    myln = jnp.max(ln_b)
    nchw = jnp.minimum(
        lax.div(jnp.maximum(myln - l0, 0) + (CHT - 1), CHT), NCH)

    for ch in range(NBUF):
        @pl.when(ch < nchw)
        def _pro(ch=ch):
            start(ch)

    @pl.when(jnp.logical_and(b == 1, h == 0))
    def _small0():
        st_w = pltpu.async_copy(wh_h, whb, sem2)
        pltpu.sync_copy(l_hs_h, lhsb)
        st_w.wait()
        lhs_v = lhsb[...]
        for j in range(H):
            hm = jnp.int32(j) >= lhs_v
            v0 = plsc.load_gather(whb, [iota, _full(j), _full(0)])
            plsc.store_scatter(scb, [iota, _full(j)], jnp.where(hm, MASK_SC, v0))
            v8 = plsc.load_gather(whb, [iota, _full(j), _full(8)])
            plsc.store_scatter(wcb, [iota, _full(j)], jnp.where(hm, MASK_WC, v8))
        pltpu.sync_copy(scb, o_sc)
        pltpu.sync_copy(wcb, o_wc)

    @pl.when(jnp.logical_and(b == 1, h == 1))
    def _small1():
        st_w = pltpu.async_copy(wh_h, whb, sem2)
        pltpu.sync_copy(cls_h, clsb)
        pltpu.sync_copy(g_sc_h, gscb)
        st_w.wait()
        gsc_v = gscb[...]
        for j in range(6):
            v = plsc.load_gather(whb, [iota, gsc_v, _full(1 + j)])
            sab[j, pl.ds(0, LANES)] = v
        for j in range(5):
            v = plsc.load_gather(clsb, [iota, _full(j)])
            wnb[j, pl.ds(0, LANES)] = v
        for w in range(4):
            cw = plsc.load_gather(gwcb, [_full(w), iota])
            for j in range(4):
                v = plsc.load_gather(whb, [iota, cw, _full(10 + j)])
                wob[w, j, pl.ds(0, LANES)] = v
        pltpu.sync_copy(sab, o_sa)
        pltpu.sync_copy(wnb, o_wn)
        pltpu.sync_copy(wob, o_wo)

    chans = []
    for w in range(4):
        c0 = plsc.load_gather(gwcb, [_full(w), _full(b)])
        chans.append((c0, c0 + 100))

    mvec = jnp.full((LANES,), jnp.float32(MASK_WV), jnp.float32)
    for ch in range(NCH):
        @pl.when(ch < nchw)
        def _work(ch=ch):
            s, d = chunk_refs(ch)
            pltpu.make_async_copy(s, d, sem).wait()
            cur = _full(ch % NBUF)

            def mloop(m, _, cur=cur, ch=ch):
                l_loc = m * LANES + iota
                nm = (l0 + ch * CHT + l_loc) >= ln_b
                for w in range(4):
                    for k in range(2):
                        vals = plsc.load_gather(gbuf, [cur, l_loc, chans[w][k]])
                        vals = jnp.where(nm, MASK_WV, vals)
                        obuf[w, ch, k, pl.ds(m * LANES, LANES)] = vals
                return 0

            lax.fori_loop(0, CHT // LANES, mloop, 0)

        @pl.when(ch >= nchw)
        def _fill(ch=ch):
            def floop(m, _, ch=ch):
                for w in range(4):
                    for k in range(2):
                        obuf[w, ch, k, pl.ds(m * LANES, LANES)] = mvec
                return 0

            lax.fori_loop(0, CHT // LANES, floop, 0)

        if ch + NBUF < NCH:
            @pl.when(ch + NBUF < nchw)
            def _nxt(ch=ch):
                start(ch + NBUF)

    writes = [
        pltpu.async_copy(obuf.at[w], o_wv.at[b * 4 + w, pl.ds(h * NCH, NCH)],
                         sem2)
        for w in range(4)
    ]
    for c in writes:
        c.wait()


def _sc_call(wn, l_n, wh, l_hs, cls, g_sc, g_wc_t):
    return pl.kernel(
        _body,
        out_type=[
            jax.ShapeDtypeStruct((B, H), jnp.float32),
            jax.ShapeDtypeStruct((6, B), jnp.float32),
            jax.ShapeDtypeStruct((5, B), jnp.float32),
            jax.ShapeDtypeStruct((B, H), jnp.float32),
            jax.ShapeDtypeStruct((4, 4, B), jnp.float32),
            jax.ShapeDtypeStruct((B * 4, L // CHT, 2, CHT), jnp.float32),
        ],
        mesh=plsc.VectorSubcoreMesh(core_axis_name="c", subcore_axis_name="s"),
        compiler_params=pltpu.CompilerParams(needs_layout_passes=False),
        scratch_types=[
            pltpu.VMEM((NBUF, CHT, CW), jnp.float32),
            pltpu.VMEM((4, NCH, 2, CHT), jnp.float32),
            pltpu.VMEM((B, H, Dh), jnp.float32),
            pltpu.VMEM((B, Dh), jnp.float32),
            pltpu.VMEM((B,), jnp.int32),
            pltpu.VMEM((B,), jnp.int32),
            pltpu.VMEM((B,), jnp.int32),
            pltpu.VMEM((4, B), jnp.int32),
            pltpu.VMEM((B, H), jnp.float32),
            pltpu.VMEM((6, B), jnp.float32),
            pltpu.VMEM((5, B), jnp.float32),
            pltpu.VMEM((B, H), jnp.float32),
            pltpu.VMEM((4, 4, B), jnp.float32),
            pltpu.SemaphoreType.DMA,
            pltpu.SemaphoreType.DMA,
        ],
    )(wn, l_n, wh, l_hs, cls, g_sc, g_wc_t)


def kernel(wemb_n, l_n, wemb_h, l_hs, cls_vec, g_sc, g_sa, g_wn, g_wc, g_wo):
    o_sc, o_sa, o_wn, o_wc, o_wo, o_wv = _sc_call(
        wemb_n,
        l_n.astype(jnp.int32),
        wemb_h,
        l_hs.astype(jnp.int32),
        cls_vec,
        g_sc.astype(jnp.int32),
        g_wc.astype(jnp.int32).T,
    )
    s_wv = (o_wv.reshape(B, 4, L // CHT, 2, CHT)
            .transpose(0, 1, 2, 4, 3)
            .reshape(B, 4, L, 2))
    return (o_sc, o_sa.T, o_wn.T, o_wc, jnp.transpose(o_wo, (2, 0, 1)), s_wv)


# speedup vs baseline: 4.2737x; 1.0284x over previous
import jax
import jax.numpy as jnp
from jax import lax
from jax.experimental import pallas as pl
from jax.experimental.pallas import tpu as pltpu
from jax.experimental.pallas import tpu_sc as plsc

B, L, H, Dn, Dh = 16, 2048, 24, 256, 100
LANES = 16
HALF = L // 2
CHT = 128
NCH = HALF // CHT
CW = 128
NBUF = 3

MASK_SC = -9999999999.0
MASK_WC = -99999999999.0
MASK_WV = -100000000000.0


def _full(v):
    return jnp.full((LANES,), v, jnp.int32)


def _body(wn, l_n_h, wh_h, l_hs_h, cls_h, g_sc_h, g_wc_h,
          o_sc, o_sa, o_wn, o_wc, o_wo, o_wv,
          gbuf, obuf, whb, clsb, lnb, lhsb, gscb, gwcb,
          scb, sab, wnb, wcb, wob, sem, sem2):
    b = lax.axis_index("s")
    h = lax.axis_index("c")
    iota = lax.iota(jnp.int32, LANES)

    def chunk_refs(ch):
        return (wn.at[b, pl.ds((2 * ch + h) * CHT, CHT), pl.ds(0, CW)],
                gbuf.at[ch % NBUF])

    def start(ch):
        s, d = chunk_refs(ch)
        pltpu.async_copy(s, d, sem)

    pltpu.sync_copy(l_n_h, lnb)
    pltpu.sync_copy(g_wc_h, gwcb)
    ln_b = plsc.load_gather(lnb, [_full(b)])

    myln = jnp.max(ln_b)
    nt = lax.div(myln + (CHT - 1), CHT)
    nchw = jnp.clip(lax.div(nt - h + 1, 2), 0, NCH)

    for ch in range(NBUF):
        @pl.when(ch < nchw)
        def _pro(ch=ch):
            start(ch)

    @pl.when(jnp.logical_and(b == 1, h == 0))
    def _small0():
        st_w = pltpu.async_copy(wh_h, whb, sem2)
        pltpu.sync_copy(l_hs_h, lhsb)
        st_w.wait()
        lhs_v = lhsb[...]
        for j in range(H):
            hm = jnp.int32(j) >= lhs_v
            v0 = plsc.load_gather(whb, [iota, _full(j), _full(0)])
            plsc.store_scatter(scb, [iota, _full(j)], jnp.where(hm, MASK_SC, v0))
            v8 = plsc.load_gather(whb, [iota, _full(j), _full(8)])
            plsc.store_scatter(wcb, [iota, _full(j)], jnp.where(hm, MASK_WC, v8))
        pltpu.sync_copy(scb, o_sc)
        pltpu.sync_copy(wcb, o_wc)

    @pl.when(jnp.logical_and(b == 1, h == 1))
    def _small1():
        st_w = pltpu.async_copy(wh_h, whb, sem2)
        pltpu.sync_copy(cls_h, clsb)
        pltpu.sync_copy(g_sc_h, gscb)
        st_w.wait()
        gsc_v = gscb[...]
        for j in range(6):
            v = plsc.load_gather(whb, [iota, gsc_v, _full(1 + j)])
            sab[j, pl.ds(0, LANES)] = v
        for j in range(5):
            v = plsc.load_gather(clsb, [iota, _full(j)])
            wnb[j, pl.ds(0, LANES)] = v
        for w in range(4):
            cw = plsc.load_gather(gwcb, [_full(w), iota])
            for j in range(4):
                v = plsc.load_gather(whb, [iota, cw, _full(10 + j)])
                wob[w, j, pl.ds(0, LANES)] = v
        pltpu.sync_copy(sab, o_sa)
        pltpu.sync_copy(wnb, o_wn)
        pltpu.sync_copy(wob, o_wo)

    chans = []
    for w in range(4):
        c0 = plsc.load_gather(gwcb, [_full(w), _full(b)])
        chans.append((c0, c0 + 100))

    mvec = jnp.full((LANES,), jnp.float32(MASK_WV), jnp.float32)
    for ch in range(NCH):
        @pl.when(ch < nchw)
        def _work(ch=ch):
            s, d = chunk_refs(ch)
            pltpu.make_async_copy(s, d, sem).wait()
            cur = _full(ch % NBUF)

            def mloop(m, _, cur=cur, ch=ch):
                l_loc = m * LANES + iota
                nm = ((2 * ch + h) * CHT + l_loc) >= ln_b
                for w in range(4):
                    for k in range(2):
                        vals = plsc.load_gather(gbuf, [cur, l_loc, chans[w][k]])
                        vals = jnp.where(nm, MASK_WV, vals)
                        obuf[w, ch, k, pl.ds(m * LANES, LANES)] = vals
                return 0

            lax.fori_loop(0, CHT // LANES, mloop, 0)

        @pl.when(ch >= nchw)
        def _fill(ch=ch):
            def floop(m, _, ch=ch):
                for w in range(4):
                    for k in range(2):
                        obuf[w, ch, k, pl.ds(m * LANES, LANES)] = mvec
                return 0

            lax.fori_loop(0, CHT // LANES, floop, 0)

        if ch + NBUF < NCH:
            @pl.when(ch + NBUF < nchw)
            def _nxt(ch=ch):
                start(ch + NBUF)

    writes = [
        pltpu.async_copy(obuf.at[w],
                         o_wv.at[b * 4 + w, pl.ds(0, NCH), h], sem2)
        for w in range(4)
    ]
    for c in writes:
        c.wait()


def _sc_call(wn, l_n, wh, l_hs, cls, g_sc, g_wc_t):
    return pl.kernel(
        _body,
        out_type=[
            jax.ShapeDtypeStruct((B, H), jnp.float32),
            jax.ShapeDtypeStruct((6, B), jnp.float32),
            jax.ShapeDtypeStruct((5, B), jnp.float32),
            jax.ShapeDtypeStruct((B, H), jnp.float32),
            jax.ShapeDtypeStruct((4, 4, B), jnp.float32),
            jax.ShapeDtypeStruct((B * 4, NCH, 2, 2, CHT), jnp.float32),
        ],
        mesh=plsc.VectorSubcoreMesh(core_axis_name="c", subcore_axis_name="s"),
        compiler_params=pltpu.CompilerParams(needs_layout_passes=False),
        scratch_types=[
            pltpu.VMEM((NBUF, CHT, CW), jnp.float32),
            pltpu.VMEM((4, NCH, 2, CHT), jnp.float32),
            pltpu.VMEM((B, H, Dh), jnp.float32),
            pltpu.VMEM((B, Dh), jnp.float32),
            pltpu.VMEM((B,), jnp.int32),
            pltpu.VMEM((B,), jnp.int32),
            pltpu.VMEM((B,), jnp.int32),
            pltpu.VMEM((4, B), jnp.int32),
            pltpu.VMEM((B, H), jnp.float32),
            pltpu.VMEM((6, B), jnp.float32),
            pltpu.VMEM((5, B), jnp.float32),
            pltpu.VMEM((B, H), jnp.float32),
            pltpu.VMEM((4, 4, B), jnp.float32),
            pltpu.SemaphoreType.DMA,
            pltpu.SemaphoreType.DMA,
        ],
    )(wn, l_n, wh, l_hs, cls, g_sc, g_wc_t)


def kernel(wemb_n, l_n, wemb_h, l_hs, cls_vec, g_sc, g_sa, g_wn, g_wc, g_wo):
    o_sc, o_sa, o_wn, o_wc, o_wo, o_wv = _sc_call(
        wemb_n,
        l_n.astype(jnp.int32),
        wemb_h,
        l_hs.astype(jnp.int32),
        cls_vec,
        g_sc.astype(jnp.int32),
        g_wc.astype(jnp.int32).T,
    )
    s_wv = (o_wv.reshape(B, 4, NCH, 2, 2, CHT)
            .transpose(0, 1, 2, 3, 5, 4)
            .reshape(B, 4, L, 2))
    return (o_sc, o_sa.T, o_wn.T, o_wc, jnp.transpose(o_wo, (2, 0, 1)), s_wv)
